# baseline (speedup 1.0000x reference)
import jax
import jax.numpy as jnp
from jax.experimental import pallas as pl

EPS_ = 0.1


def _zono_body(x_ref, o_ref):
    bb, hwb = x_ref.shape[0], x_ref.shape[2]
    m_dim = o_ref.shape[3]
    xv = x_ref[:, 0:1, :]
    lo = xv < EPS_
    hi = xv > 1.0 - EPS_
    center = jnp.where(lo, (xv + EPS_) * 0.5,
             jnp.where(hi, (xv + 1.0 - EPS_) * 0.5, xv))
    err = jnp.where(lo, (EPS_ + xv) * 0.5,
          jnp.where(hi, (1.0 - xv + EPS_) * 0.5, jnp.full_like(xv, EPS_)))
    def _split(v):
        v_hi = jax.lax.convert_element_type(
            jax.lax.convert_element_type(v, jnp.bfloat16), jnp.float32)
        return v_hi, v - v_hi

    c_hi, c_lo = _split(center)
    e_hi, e_lo = _split(err)
    a4 = jnp.swapaxes(
        jnp.concatenate([c_hi, c_lo, e_hi, e_lo], axis=1), 1, 2)
    kr = jax.lax.broadcasted_iota(jnp.int32, (bb, 4, 128), 1)
    er = jax.lax.broadcasted_iota(jnp.int32, (bb, 4, 128), 2)
    bmix = jnp.where(kr < 2, (er == 0).astype(xv.dtype),
                     (er >= 1).astype(xv.dtype))
    omix = jax.lax.dot_general(
        a4, bmix, (((2,), (1,)), ((0,), (0,))),
        preferred_element_type=jnp.float32)
    oerr = jax.lax.dot_general(
        a4[:, :, 2:4], jnp.ones((bb, 2, 128), xv.dtype),
        (((2,), (1,)), ((0,), (0,))),
        preferred_element_type=jnp.float32)

    r0 = jax.lax.broadcasted_iota(jnp.int32, (bb, hwb, 128), 1)
    e0 = jax.lax.broadcasted_iota(jnp.int32, (bb, hwb, 128), 2)
    keep = (e0 == r0 + 1) | (e0 == 0)
    o_ref[:, :, 0, 0:128] = jnp.where(keep, omix, 0.0)
    o_ref[:, :, 0, 128:m_dim] = jnp.zeros((bb, hwb, m_dim - 128), xv.dtype)
    for j in range(1, m_dim // 128):
        rs = 128 * j - 8
        rl = min(136, hwb - rs)
        if rl <= 0:
            break
        rr = jax.lax.broadcasted_iota(jnp.int32, (bb, rl, 128), 1) + rs
        ee = jax.lax.broadcasted_iota(jnp.int32, (bb, rl, 128), 2) + 128 * j
        w = jnp.where(ee == rr + 1, oerr[:, rs:rs + rl, :], 0.0)
        o_ref[:, rs:rs + rl, 0, 128 * j:128 * (j + 1)] = w


def kernel(x):
    B, C, H, W = x.shape
    P = C * H * W
    E = 1 + P
    M = 896
    BB = 4
    x3 = x.reshape(B, 1, P)
    out4 = pl.pallas_call(
        _zono_body,
        grid=(B // BB,),
        in_specs=[pl.BlockSpec((BB, 1, P), lambda b: (b, 0, 0))],
        out_specs=pl.BlockSpec((BB, P, 1, M), lambda b: (b, 0, 0, 0)),
        out_shape=jax.ShapeDtypeStruct((B, P, 1, M), x.dtype),
    )(x3)
    return out4[:, :, :, :E].reshape(B, H, W, 1, E).transpose(0, 4, 3, 1, 2)


# speedup vs baseline: 1.0485x; 1.0485x over previous
import jax
import jax.numpy as jnp
from jax.experimental import pallas as pl

EPS_ = 0.1


def _zono_body(x_ref, o_ref):
    bb, hwb = x_ref.shape[0], x_ref.shape[2]
    m_dim = o_ref.shape[3]
    xv = x_ref[:, 0:1, :]
    lo = xv < EPS_
    hi = xv > 1.0 - EPS_
    center = jnp.where(lo, (xv + EPS_) * 0.5,
             jnp.where(hi, (xv + 1.0 - EPS_) * 0.5, xv))
    err = jnp.where(lo, (EPS_ + xv) * 0.5,
          jnp.where(hi, (1.0 - xv + EPS_) * 0.5, jnp.full_like(xv, EPS_)))
    def _split(v):
        v_hi = jax.lax.convert_element_type(
            jax.lax.convert_element_type(v, jnp.bfloat16), jnp.float32)
        return v_hi, v - v_hi

    c_hi, c_lo = _split(center)
    e_hi, e_lo = _split(err)
    a4 = jnp.swapaxes(
        jnp.concatenate([c_hi, c_lo, e_hi, e_lo], axis=1), 1, 2)
    kr = jax.lax.broadcasted_iota(jnp.int32, (bb, 4, m_dim), 1)
    er = jax.lax.broadcasted_iota(jnp.int32, (bb, 4, m_dim), 2)
    bm = jnp.where(kr < 2, (er == 0).astype(xv.dtype),
                   (er >= 1).astype(xv.dtype))
    full = jax.lax.dot_general(
        a4, bm, (((2,), (1,)), ((0,), (0,))),
        preferred_element_type=jnp.float32)
    r0 = jax.lax.broadcasted_iota(jnp.int32, (bb, hwb, m_dim), 1)
    e0 = jax.lax.broadcasted_iota(jnp.int32, (bb, hwb, m_dim), 2)
    keep = (e0 == r0 + 1) | (e0 == 0)
    o_ref[:, :, 0, :] = jnp.where(keep, full, 0.0)


def kernel(x):
    B, C, H, W = x.shape
    P = C * H * W
    E = 1 + P
    M = 896
    BB = 4
    x3 = x.reshape(B, 1, P)
    out4 = pl.pallas_call(
        _zono_body,
        grid=(B // BB,),
        in_specs=[pl.BlockSpec((BB, 1, P), lambda b: (b, 0, 0))],
        out_specs=pl.BlockSpec((BB, P, 1, M), lambda b: (b, 0, 0, 0)),
        out_shape=jax.ShapeDtypeStruct((B, P, 1, M), x.dtype),
    )(x3)
    return out4[:, :, :, :E].reshape(B, H, W, 1, E).transpose(0, 4, 3, 1, 2)
